# baseline (speedup 1.0000x reference)
import functools

import jax
import jax.numpy as jnp
from jax import lax
from jax.experimental import pallas as pl
from jax.experimental.pallas import tpu as pltpu
from jax.experimental.pallas import tpu_sc as plsc


@functools.lru_cache(maxsize=None)
def _make_sc_gather(Bc, D, NC, NS, chunk_off, CH=128):
    NW = NC * NS
    b_per_w = Bc // NW
    n_ch = b_per_w // CH
    mesh = plsc.VectorSubcoreMesh(core_axis_name="c", subcore_axis_name="s")

    @functools.partial(
        pl.kernel,
        mesh=mesh,
        out_type=[
            jax.ShapeDtypeStruct((Bc, D), jnp.float32),
            jax.ShapeDtypeStruct((Bc, D), jnp.float32),
        ],
        scratch_types=[
            pltpu.VMEM((b_per_w,), jnp.int32),
            pltpu.VMEM((b_per_w,), jnp.int32),
            pltpu.VMEM((b_per_w, D), jnp.float32),
            pltpu.VMEM((b_per_w, D), jnp.float32),
            pltpu.SemaphoreType.DMA,
            pltpu.SemaphoreType.DMA,
            pltpu.SemaphoreType.DMA,
        ],
    )
    def gather2(uidx_hbm, iidx_hbm, ut_hbm, it_hbm, out_u, out_i,
                idx_u, idx_i, rows_u, rows_i, sem_i, sem_g, sem_s):
        wid = lax.axis_index("s") * NC + lax.axis_index("c")
        base = wid * b_per_w
        ci = [pltpu.async_copy(h.at[pl.ds(chunk_off + base, b_per_w)], v, sem_i)
              for h, v in ((uidx_hbm, idx_u), (iidx_hbm, idx_i))]
        for c in ci:
            c.wait()
        gs = []
        for tbl, iv, rv in ((ut_hbm, idx_u, rows_u), (it_hbm, idx_i, rows_i)):
            gs.append([
                pltpu.async_copy(tbl.at[iv.at[pl.ds(j * CH, CH)]],
                                 rv.at[pl.ds(j * CH, CH)], sem_g)
                for j in range(n_ch)
            ])
        sts = []
        for (tbl_gs, rv, out) in ((gs[0], rows_u, out_u), (gs[1], rows_i, out_i)):
            for c in tbl_gs:
                c.wait()
            sts.append(pltpu.async_copy(rv, out.at[pl.ds(base, b_per_w)], sem_s))
        for s in sts:
            s.wait()

    return gather2


def _dense_body(ue_ref, ie_ref, wt_ref, b_ref, out_ref):
    wt = wt_ref[...].astype(jnp.bfloat16)
    dn = (((1,), (1,)), ((), ()))
    ut = lax.dot_general(wt, ue_ref[...].astype(jnp.bfloat16), dn,
                         preferred_element_type=jnp.float32)
    vt = lax.dot_general(wt, ie_ref[...].astype(jnp.bfloat16), dn,
                         preferred_element_type=jnp.float32)
    bb = b_ref[...]
    u = jnp.maximum(ut + bb, 0.0)
    v = jnp.maximum(vt + bb, 0.0)
    dots = jnp.sum(u * v, axis=0, keepdims=True)
    uu = jnp.sum(u * u, axis=0, keepdims=True)
    vv = jnp.sum(v * v, axis=0, keepdims=True)
    denom = jnp.sqrt(jnp.maximum(uu, 1e-16) * jnp.maximum(vv, 1e-16))
    out_ref[...] = (dots / denom)[None]


def kernel(user_indices, item_indices, user_table, item_table, W, b):
    B = user_indices.shape[0]
    D = user_table.shape[1]
    N = W.shape[1]
    NP = (N + 127) // 128 * 128
    NCHUNK = 2
    Bc = B // NCHUNK
    BM = 1024

    info = plsc.get_sparse_core_info()
    NC, NS = info.num_cores, info.num_subcores

    uidx = user_indices.astype(jnp.int32)
    iidx = item_indices.astype(jnp.int32)

    Wt = jnp.pad(W, ((0, 0), (0, NP - N))).T
    bp = jnp.pad(b, (0, NP - N)).reshape(NP, 1)

    parts = []
    for c in range(NCHUNK):
        ue, ie = _make_sc_gather(Bc, D, NC, NS, c * Bc)(
            uidx, iidx, user_table, item_table)
        parts.append(pl.pallas_call(
            _dense_body,
            grid=(Bc // BM,),
            in_specs=[
                pl.BlockSpec((BM, D), lambda i: (i, 0)),
                pl.BlockSpec((BM, D), lambda i: (i, 0)),
                pl.BlockSpec((NP, D), lambda i: (0, 0)),
                pl.BlockSpec((NP, 1), lambda i: (0, 0)),
            ],
            out_specs=pl.BlockSpec((1, 1, BM), lambda i: (i, 0, 0)),
            out_shape=jax.ShapeDtypeStruct((Bc // BM, 1, BM), jnp.float32),
        )(ue, ie, Wt, bp))
    return jnp.concatenate(parts, axis=0).reshape(B)


# speedup vs baseline: 1.0465x; 1.0465x over previous
import functools

import jax
import jax.numpy as jnp
from jax import lax
from jax.experimental import pallas as pl
from jax.experimental.pallas import tpu as pltpu
from jax.experimental.pallas import tpu_sc as plsc


@functools.lru_cache(maxsize=None)
def _make_sc_gather(Bc, D, NC, NS, chunk_off, CH=128):
    NW = NC * NS
    b_per_w = Bc // NW
    n_ch = b_per_w // CH
    mesh = plsc.VectorSubcoreMesh(core_axis_name="c", subcore_axis_name="s")

    @functools.partial(
        pl.kernel,
        mesh=mesh,
        out_type=[
            jax.ShapeDtypeStruct((Bc, D), jnp.float32),
            jax.ShapeDtypeStruct((Bc, D), jnp.float32),
        ],
        scratch_types=[
            pltpu.VMEM((b_per_w,), jnp.int32),
            pltpu.VMEM((b_per_w,), jnp.int32),
            pltpu.VMEM((b_per_w, D), jnp.float32),
            pltpu.VMEM((b_per_w, D), jnp.float32),
            pltpu.SemaphoreType.DMA,
            pltpu.SemaphoreType.DMA,
            pltpu.SemaphoreType.DMA,
        ],
    )
    def gather2(uidx_hbm, iidx_hbm, ut_hbm, it_hbm, out_u, out_i,
                idx_u, idx_i, rows_u, rows_i, sem_i, sem_g, sem_s):
        wid = lax.axis_index("s") * NC + lax.axis_index("c")
        base = wid * b_per_w
        ci = [pltpu.async_copy(h.at[pl.ds(chunk_off + base, b_per_w)], v, sem_i)
              for h, v in ((uidx_hbm, idx_u), (iidx_hbm, idx_i))]
        for c in ci:
            c.wait()
        gs = []
        for tbl, iv, rv in ((ut_hbm, idx_u, rows_u), (it_hbm, idx_i, rows_i)):
            gs.append([
                pltpu.async_copy(tbl.at[iv.at[pl.ds(j * CH, CH)]],
                                 rv.at[pl.ds(j * CH, CH)], sem_g)
                for j in range(n_ch)
            ])
        sts = []
        for (tbl_gs, rv, out) in ((gs[0], rows_u, out_u), (gs[1], rows_i, out_i)):
            for c in tbl_gs:
                c.wait()
            sts.append(pltpu.async_copy(rv, out.at[pl.ds(base, b_per_w)], sem_s))
        for s in sts:
            s.wait()

    return gather2


def _dense_body(ue_ref, ie_ref, wt_ref, b_ref, out_ref):
    wt = wt_ref[...].astype(jnp.bfloat16)
    dn = (((1,), (1,)), ((), ()))
    ut = lax.dot_general(wt, ue_ref[...].astype(jnp.bfloat16), dn,
                         preferred_element_type=jnp.float32)
    vt = lax.dot_general(wt, ie_ref[...].astype(jnp.bfloat16), dn,
                         preferred_element_type=jnp.float32)
    bb = b_ref[...]
    u = jnp.maximum(ut + bb, 0.0)
    v = jnp.maximum(vt + bb, 0.0)
    dots = jnp.sum(u * v, axis=0, keepdims=True)
    uu = jnp.sum(u * u, axis=0, keepdims=True)
    vv = jnp.sum(v * v, axis=0, keepdims=True)
    denom = jnp.sqrt(jnp.maximum(uu, 1e-16) * jnp.maximum(vv, 1e-16))
    out_ref[...] = (dots / denom)[None]


def kernel(user_indices, item_indices, user_table, item_table, W, b):
    B = user_indices.shape[0]
    D = user_table.shape[1]
    N = W.shape[1]
    NP = (N + 127) // 128 * 128
    NCHUNK = 2
    Bc = B // NCHUNK
    BM = 4096

    info = plsc.get_sparse_core_info()
    NC, NS = info.num_cores, info.num_subcores

    uidx = user_indices.astype(jnp.int32)
    iidx = item_indices.astype(jnp.int32)

    Wt = jnp.pad(W, ((0, 0), (0, NP - N))).T
    bp = jnp.pad(b, (0, NP - N)).reshape(NP, 1)

    parts = []
    for c in range(NCHUNK):
        ue, ie = _make_sc_gather(Bc, D, NC, NS, c * Bc)(
            uidx, iidx, user_table, item_table)
        parts.append(pl.pallas_call(
            _dense_body,
            grid=(Bc // BM,),
            in_specs=[
                pl.BlockSpec((BM, D), lambda i: (i, 0)),
                pl.BlockSpec((BM, D), lambda i: (i, 0)),
                pl.BlockSpec((NP, D), lambda i: (0, 0)),
                pl.BlockSpec((NP, 1), lambda i: (0, 0)),
            ],
            out_specs=pl.BlockSpec((1, 1, BM), lambda i: (i, 0, 0)),
            out_shape=jax.ShapeDtypeStruct((Bc // BM, 1, BM), jnp.float32),
        )(ue, ie, Wt, bp))
    return jnp.concatenate(parts, axis=0).reshape(B)
